# baseline (speedup 1.0000x reference)
import functools

import jax
import jax.numpy as jnp
from jax import lax
from jax.experimental import pallas as pl
from jax.experimental.pallas import tpu as pltpu
from jax.experimental.pallas import tpu_sc as plsc

N = 10000
E = 320000
D = 128
DH = 128
DW = 16
NC, NS = 2, 16
NW = NC * NS
C = 64
E2 = 327680
NCH1 = E2 // (NW * C)
NCH2 = E2 // (NW * C)
NP = 10112
RPS = NP // NS

_MESH = plsc.VectorSubcoreMesh(core_axis_name="c", subcore_axis_name="s")


@functools.partial(
    pl.kernel,
    out_type=[
        jax.ShapeDtypeStruct((NC, NP, DH), jnp.float32),
        jax.ShapeDtypeStruct((NC, NP, DW), jnp.float32),
    ],
    mesh=_MESH,
    compiler_params=pltpu.CompilerParams(use_tc_tiling_on_sc=False),
    scratch_types=[
        pltpu.VMEM((NCH1, C), jnp.int32),
        pltpu.VMEM((NCH1, C), jnp.int32),
        pltpu.VMEM((C, DH), jnp.float32),
        pltpu.VMEM((C, DH), jnp.float32),
        pltpu.VMEM((C, DW), jnp.float32),
        pltpu.VMEM_SHARED((NP, DH), jnp.float32),
        pltpu.VMEM_SHARED((NP, DW), jnp.float32),
        pltpu.SemaphoreType.DMA,
        pltpu.SemaphoreType.DMA,
        pltpu.SemaphoreType.DMA,
    ],
)
def _sc_agg_wide(src_hbm, dst_hbm, y_hbm, zh_hbm, zw_hbm,
                 sum_out, cnt_out,
                 src_v, dst_v, rows_a, rows_b, ones_v,
                 acc_sh, cnt_sh,
                 sem_a, sem_b, sem_c):
    cid = lax.axis_index("c")
    sid = lax.axis_index("s")
    wid = cid * NS + sid

    pltpu.sync_copy(zh_hbm.at[pl.ds(sid * RPS, RPS)],
                    acc_sh.at[pl.ds(sid * RPS, RPS)])
    pltpu.sync_copy(zw_hbm.at[pl.ds(sid * RPS, RPS)],
                    cnt_sh.at[pl.ds(sid * RPS, RPS)])

    pltpu.sync_copy(src_hbm.at[wid], src_v)
    pltpu.sync_copy(dst_hbm.at[wid], dst_v)

    @pl.loop(0, C)
    def _fill_ones(i):
        ones_v[i, :] = jnp.ones((DW,), jnp.float32)

    plsc.subcore_barrier()

    pltpu.async_copy(y_hbm.at[src_v.at[0]], rows_a, sem_a)

    @pl.loop(0, NCH1 // 2)
    def _pair(j):
        i0 = 2 * j
        i1 = 2 * j + 1
        db = pltpu.async_copy(y_hbm.at[src_v.at[i1]], rows_b, sem_b)
        pltpu.make_async_copy(y_hbm.at[src_v.at[i0]], rows_a,
                              sem_a).wait()
        pltpu.sync_copy(rows_a, acc_sh.at[dst_v.at[i0]], add=True)

        pltpu.async_copy(ones_v, cnt_sh.at[dst_v.at[i0]], sem_c, add=True)

        @pl.when(j < NCH1 // 2 - 1)
        def _next():
            pltpu.async_copy(y_hbm.at[src_v.at[i0 + 2]], rows_a,
                             sem_a)

        db.wait()
        pltpu.sync_copy(rows_b, acc_sh.at[dst_v.at[i1]], add=True)

        pltpu.async_copy(ones_v, cnt_sh.at[dst_v.at[i1]], sem_c, add=True)

    @pl.loop(0, NCH1)
    def _drain(i):
        pltpu.make_async_copy(ones_v, cnt_sh.at[dst_v.at[0]], sem_c).wait()

    plsc.subcore_barrier()

    pltpu.sync_copy(acc_sh.at[pl.ds(sid * RPS, RPS)],
                    sum_out.at[cid, pl.ds(sid * RPS, RPS)])

    pltpu.sync_copy(cnt_sh.at[pl.ds(sid * RPS, RPS)],
                    cnt_out.at[cid, pl.ds(sid * RPS, RPS)])


@functools.partial(
    pl.kernel,
    out_type=jax.ShapeDtypeStruct((NC, NP, DW), jnp.float32),
    mesh=_MESH,
    compiler_params=pltpu.CompilerParams(use_tc_tiling_on_sc=False),
    scratch_types=[
        pltpu.VMEM((NCH2, C), jnp.int32),
        pltpu.VMEM((NCH2, C), jnp.int32),
        pltpu.VMEM((C, DW), jnp.float32),
        pltpu.VMEM((C, DW), jnp.float32),
        pltpu.VMEM_SHARED((NP, DW), jnp.float32),
        pltpu.SemaphoreType.DMA,
        pltpu.SemaphoreType.DMA,
    ],
)
def _sc_agg_narrow(src_hbm, dst_hbm, t_hbm, zw_hbm, sum_out,
                   src_v, dst_v, rows_a, rows_b, acc_sh, sem_a, sem_b):
    cid = lax.axis_index("c")
    sid = lax.axis_index("s")
    wid = cid * NS + sid

    pltpu.sync_copy(zw_hbm.at[pl.ds(sid * RPS, RPS)],
                    acc_sh.at[pl.ds(sid * RPS, RPS)])
    pltpu.sync_copy(src_hbm.at[wid], src_v)
    pltpu.sync_copy(dst_hbm.at[wid], dst_v)

    plsc.subcore_barrier()

    pltpu.async_copy(t_hbm.at[src_v.at[0]], rows_a, sem_a)

    @pl.loop(0, NCH2 // 2)
    def _pair(j):
        i0 = 2 * j
        i1 = 2 * j + 1
        db = pltpu.async_copy(t_hbm.at[src_v.at[i1]], rows_b, sem_b)
        pltpu.make_async_copy(t_hbm.at[src_v.at[i0]], rows_a, sem_a).wait()
        pltpu.sync_copy(rows_a, acc_sh.at[dst_v.at[i0]], add=True)

        @pl.when(j < NCH2 // 2 - 1)
        def _next():
            pltpu.async_copy(t_hbm.at[src_v.at[i0 + 2]], rows_a, sem_a)

        db.wait()
        pltpu.sync_copy(rows_b, acc_sh.at[dst_v.at[i1]], add=True)

    plsc.subcore_barrier()

    pltpu.sync_copy(acc_sh.at[pl.ds(sid * RPS, RPS)],
                    sum_out.at[cid, pl.ds(sid * RPS, RPS)])


_BLK = 1000


def _tc_pre(x, W1l, W1r, b1l, b1r):
    def body(x_ref, wl_ref, wr_ref, bl_ref, br_ref, y_ref, u_ref):
        xb = x_ref[...]
        y_ref[...] = lax.dot_general(xb, wl_ref[...], (((1,), (1,)), ((), ())),
                                     preferred_element_type=jnp.float32)
        u_ref[...] = (lax.dot_general(xb, wr_ref[...], (((1,), (1,)), ((), ())),
                                      preferred_element_type=jnp.float32)
                      + bl_ref[...] + br_ref[...])

    full = pl.BlockSpec((D, D), lambda i: (0, 0))
    bias = pl.BlockSpec((1, D), lambda i: (0, 0))
    return pl.pallas_call(
        body,
        grid=(N // _BLK,),
        in_specs=[pl.BlockSpec((_BLK, D), lambda i: (i, 0)), full, full,
                  bias, bias],
        out_specs=[pl.BlockSpec((_BLK, D), lambda i: (i, 0)),
                   pl.BlockSpec((_BLK, D), lambda i: (i, 0))],
        out_shape=[jax.ShapeDtypeStruct((N, D), jnp.float32),
                   jax.ShapeDtypeStruct((N, D), jnp.float32)],
    )(x, W1l, W1r, b1l.reshape(1, D), b1r.reshape(1, D))


def _tc_mid(s1, cnt, u, W2l, W2r, Wlin):
    def body(s_ref, c_ref, u_ref, w2l_ref, w2r_ref, wlin_ref, t_ref):
        s = s_ref[0] + s_ref[1]
        cntv = jnp.maximum(c_ref[0, :, 0:1] + c_ref[1, :, 0:1], 1.0)
        h = jnp.maximum(s / cntv + u_ref[...], 0.0)
        wl = wlin_ref[...]
        m = lax.dot_general(wl, w2l_ref[...], (((1,), (0,)), ((), ())),
                            preferred_element_type=jnp.float32)
        r = lax.dot_general(wl, w2r_ref[...], (((1,), (0,)), ((), ())),
                            preferred_element_type=jnp.float32)
        g = jnp.concatenate([m, r, jnp.zeros((DW - 4, D), jnp.float32)], axis=0)
        t_ref[...] = lax.dot_general(h, g, (((1,), (1,)), ((), ())),
                                     preferred_element_type=jnp.float32)

    full = pl.BlockSpec((D, D), lambda i: (0, 0))
    return pl.pallas_call(
        body,
        grid=(N // _BLK,),
        in_specs=[pl.BlockSpec((NC, _BLK, DH), lambda i: (0, i, 0)),
                  pl.BlockSpec((NC, _BLK, DW), lambda i: (0, i, 0)),
                  pl.BlockSpec((_BLK, D), lambda i: (i, 0)),
                  full, full,
                  pl.BlockSpec((2, D), lambda i: (0, 0))],
        out_specs=pl.BlockSpec((_BLK, DW), lambda i: (i, 0)),
        out_shape=jax.ShapeDtypeStruct((N, DW), jnp.float32),
    )(s1, cnt, u, W2l, W2r, Wlin)


def _tc_post(s2, cnt, t, Wlin, b2l, b2r, blin):
    def body(s_ref, c_ref, t_ref, wlin_ref, bl_ref, br_ref, blin_ref, o_ref):
        s = s_ref[0] + s_ref[1]
        cntv = jnp.maximum(c_ref[0, :, 0:1] + c_ref[1, :, 0:1], 1.0)
        agg = s[:, 0:2] / cntv
        b2 = bl_ref[...] + br_ref[...]
        cconst = lax.dot_general(b2, wlin_ref[...], (((1,), (1,)), ((), ())),
                                 preferred_element_type=jnp.float32)
        o_ref[...] = agg + t_ref[:, 2:4] + cconst + blin_ref[...]

    return pl.pallas_call(
        body,
        grid=(N // _BLK,),
        in_specs=[pl.BlockSpec((NC, _BLK, DW), lambda i: (0, i, 0)),
                  pl.BlockSpec((NC, _BLK, DW), lambda i: (0, i, 0)),
                  pl.BlockSpec((_BLK, DW), lambda i: (i, 0)),
                  pl.BlockSpec((2, D), lambda i: (0, 0)),
                  pl.BlockSpec((1, D), lambda i: (0, 0)),
                  pl.BlockSpec((1, D), lambda i: (0, 0)),
                  pl.BlockSpec((1, 2), lambda i: (0, 0))],
        out_specs=pl.BlockSpec((_BLK, 2), lambda i: (i, 0)),
        out_shape=jax.ShapeDtypeStruct((N, 2), jnp.float32),
    )(s2, cnt, t, Wlin, b2l.reshape(1, D), b2r.reshape(1, D),
      blin.reshape(1, 2))


def kernel(x, edge_index, W1l, b1l, W1r, b1r, W2l, b2l, W2r, b2r, Wlin, blin):
    pad = E2 - E
    srcp = jnp.concatenate([edge_index[0], jnp.zeros((pad,), jnp.int32)])
    dstp = jnp.concatenate([edge_index[1], jnp.full((pad,), N, jnp.int32)])
    src1 = srcp.reshape(NW, NCH1, C)
    dst1 = dstp.reshape(NW, NCH1, C)
    src2 = srcp.reshape(NW, NCH2, C)
    dst2 = dstp.reshape(NW, NCH2, C)
    zh = jnp.zeros((NP, DH), jnp.float32)
    zw = jnp.zeros((NP, DW), jnp.float32)

    y, u = _tc_pre(x, W1l, W1r, b1l, b1r)
    s1, cnt = _sc_agg_wide(src1, dst1, y, zh, zw)
    t = _tc_mid(s1, cnt, u, W2l, W2r, Wlin)
    s2 = _sc_agg_narrow(src2, dst2, t, zw)
    return _tc_post(s2, cnt, t, Wlin, b2l, b2r, blin)


# speedup vs baseline: 1.2403x; 1.2403x over previous
import functools

import jax
import jax.numpy as jnp
from jax import lax
from jax.experimental import pallas as pl
from jax.experimental.pallas import tpu as pltpu
from jax.experimental.pallas import tpu_sc as plsc

N = 10000
E = 320000
D = 128
DH = 64
DW = 16
NC, NS = 2, 16
NW = NC * NS
C = 256
E2 = 327680
NCH1 = E2 // (NS * C)
NCH2 = E2 // (NW * C)
NP = 10112
RPS = NP // NS

_MESH = plsc.VectorSubcoreMesh(core_axis_name="c", subcore_axis_name="s")


@functools.partial(
    pl.kernel,
    out_type=[
        jax.ShapeDtypeStruct((NC, NP, DH), jnp.float32),
        jax.ShapeDtypeStruct((NP, DW), jnp.float32),
    ],
    mesh=_MESH,
    compiler_params=pltpu.CompilerParams(use_tc_tiling_on_sc=False),
    scratch_types=[
        pltpu.VMEM((NCH1, C), jnp.int32),
        pltpu.VMEM((NCH1, C), jnp.int32),
        pltpu.VMEM((C, DH), jnp.float32),
        pltpu.VMEM((C, DH), jnp.float32),
        pltpu.VMEM((C, DW), jnp.float32),
        pltpu.VMEM_SHARED((NP, DH), jnp.float32),
        pltpu.VMEM_SHARED((NP, DW), jnp.float32),
        pltpu.SemaphoreType.DMA,
        pltpu.SemaphoreType.DMA,
        pltpu.SemaphoreType.DMA,
    ],
)
def _sc_agg_wide(src_hbm, dst_hbm, y_hbm, zh_hbm, zw_hbm,
                 sum_out, cnt_out,
                 src_v, dst_v, rows_a, rows_b, ones_v,
                 acc_sh, cnt_sh,
                 sem_a, sem_b, sem_c):
    cid = lax.axis_index("c")
    sid = lax.axis_index("s")

    pltpu.sync_copy(zh_hbm.at[pl.ds(sid * RPS, RPS)],
                    acc_sh.at[pl.ds(sid * RPS, RPS)])
    pltpu.sync_copy(zw_hbm.at[pl.ds(sid * RPS, RPS)],
                    cnt_sh.at[pl.ds(sid * RPS, RPS)])

    pltpu.sync_copy(src_hbm.at[sid], src_v)
    pltpu.sync_copy(dst_hbm.at[sid], dst_v)

    @pl.loop(0, C)
    def _fill_ones(i):
        ones_v[i, :] = jnp.ones((DW,), jnp.float32)

    plsc.subcore_barrier()

    pltpu.async_copy(y_hbm.at[cid].at[src_v.at[0]], rows_a, sem_a)

    @pl.loop(0, NCH1 // 2)
    def _pair(j):
        i0 = 2 * j
        i1 = 2 * j + 1
        db = pltpu.async_copy(y_hbm.at[cid].at[src_v.at[i1]], rows_b, sem_b)
        pltpu.make_async_copy(y_hbm.at[cid].at[src_v.at[i0]], rows_a,
                              sem_a).wait()
        pltpu.sync_copy(rows_a, acc_sh.at[dst_v.at[i0]], add=True)

        @pl.when(cid == 0)
        def _count0():
            pltpu.async_copy(ones_v, cnt_sh.at[dst_v.at[i0]], sem_c,
                             add=True)

        @pl.when(j < NCH1 // 2 - 1)
        def _next():
            pltpu.async_copy(y_hbm.at[cid].at[src_v.at[i0 + 2]], rows_a,
                             sem_a)

        db.wait()
        pltpu.sync_copy(rows_b, acc_sh.at[dst_v.at[i1]], add=True)

        @pl.when(cid == 0)
        def _count1():
            pltpu.async_copy(ones_v, cnt_sh.at[dst_v.at[i1]], sem_c,
                             add=True)

    @pl.when(cid == 0)
    def _drain_counts():
        @pl.loop(0, NCH1)
        def _drain(i):
            pltpu.make_async_copy(ones_v, cnt_sh.at[dst_v.at[0]],
                                  sem_c).wait()

    plsc.subcore_barrier()

    pltpu.sync_copy(acc_sh.at[pl.ds(sid * RPS, RPS)],
                    sum_out.at[cid, pl.ds(sid * RPS, RPS)])

    @pl.when(cid == 0)
    def _dump_cnt():
        pltpu.sync_copy(cnt_sh.at[pl.ds(sid * RPS, RPS)],
                        cnt_out.at[pl.ds(sid * RPS, RPS)])


@functools.partial(
    pl.kernel,
    out_type=jax.ShapeDtypeStruct((NC, NP, DW), jnp.float32),
    mesh=_MESH,
    compiler_params=pltpu.CompilerParams(use_tc_tiling_on_sc=False),
    scratch_types=[
        pltpu.VMEM((NCH2, C), jnp.int32),
        pltpu.VMEM((NCH2, C), jnp.int32),
        pltpu.VMEM((C, DW), jnp.float32),
        pltpu.VMEM((C, DW), jnp.float32),
        pltpu.VMEM_SHARED((NP, DW), jnp.float32),
        pltpu.SemaphoreType.DMA,
        pltpu.SemaphoreType.DMA,
    ],
)
def _sc_agg_narrow(src_hbm, dst_hbm, t_hbm, zw_hbm, sum_out,
                   src_v, dst_v, rows_a, rows_b, acc_sh, sem_a, sem_b):
    cid = lax.axis_index("c")
    sid = lax.axis_index("s")
    wid = cid * NS + sid

    pltpu.sync_copy(zw_hbm.at[pl.ds(sid * RPS, RPS)],
                    acc_sh.at[pl.ds(sid * RPS, RPS)])
    pltpu.sync_copy(src_hbm.at[wid], src_v)
    pltpu.sync_copy(dst_hbm.at[wid], dst_v)

    plsc.subcore_barrier()

    pltpu.async_copy(t_hbm.at[src_v.at[0]], rows_a, sem_a)

    @pl.loop(0, NCH2 // 2)
    def _pair(j):
        i0 = 2 * j
        i1 = 2 * j + 1
        db = pltpu.async_copy(t_hbm.at[src_v.at[i1]], rows_b, sem_b)
        pltpu.make_async_copy(t_hbm.at[src_v.at[i0]], rows_a, sem_a).wait()
        pltpu.sync_copy(rows_a, acc_sh.at[dst_v.at[i0]], add=True)

        @pl.when(j < NCH2 // 2 - 1)
        def _next():
            pltpu.async_copy(t_hbm.at[src_v.at[i0 + 2]], rows_a, sem_a)

        db.wait()
        pltpu.sync_copy(rows_b, acc_sh.at[dst_v.at[i1]], add=True)

    plsc.subcore_barrier()

    pltpu.sync_copy(acc_sh.at[pl.ds(sid * RPS, RPS)],
                    sum_out.at[cid, pl.ds(sid * RPS, RPS)])


_BLK = 1000


def _tc_pre(x, W1l, W1r, b1l, b1r):
    def body(x_ref, wl_ref, wr_ref, bl_ref, br_ref, y_ref, u_ref):
        xb = x_ref[...]
        y = lax.dot_general(xb, wl_ref[...], (((1,), (1,)), ((), ())),
                            preferred_element_type=jnp.float32)
        y_ref[0, :, :] = y[:, :DH]
        y_ref[1, :, :] = y[:, DH:]
        u_ref[...] = (lax.dot_general(xb, wr_ref[...], (((1,), (1,)), ((), ())),
                                      preferred_element_type=jnp.float32)
                      + bl_ref[...] + br_ref[...])

    full = pl.BlockSpec((D, D), lambda i: (0, 0))
    bias = pl.BlockSpec((1, D), lambda i: (0, 0))
    return pl.pallas_call(
        body,
        grid=(N // _BLK,),
        in_specs=[pl.BlockSpec((_BLK, D), lambda i: (i, 0)), full, full,
                  bias, bias],
        out_specs=[pl.BlockSpec((NC, _BLK, DH), lambda i: (0, i, 0)),
                   pl.BlockSpec((_BLK, D), lambda i: (i, 0))],
        out_shape=[jax.ShapeDtypeStruct((NC, N, DH), jnp.float32),
                   jax.ShapeDtypeStruct((N, D), jnp.float32)],
    )(x, W1l, W1r, b1l.reshape(1, D), b1r.reshape(1, D))


def _tc_mid(s1, cnt, u, W2l, W2r, Wlin):
    def body(s_ref, c_ref, u_ref, w2l_ref, w2r_ref, wlin_ref, t_ref):
        s = jnp.concatenate([s_ref[0], s_ref[1]], axis=1)
        cntv = jnp.maximum(c_ref[:, 0:1], 1.0)
        h = jnp.maximum(s / cntv + u_ref[...], 0.0)
        wl = wlin_ref[...]
        m = lax.dot_general(wl, w2l_ref[...], (((1,), (0,)), ((), ())),
                            preferred_element_type=jnp.float32)
        r = lax.dot_general(wl, w2r_ref[...], (((1,), (0,)), ((), ())),
                            preferred_element_type=jnp.float32)
        g = jnp.concatenate([m, r, jnp.zeros((DW - 4, D), jnp.float32)], axis=0)
        t_ref[...] = lax.dot_general(h, g, (((1,), (1,)), ((), ())),
                                     preferred_element_type=jnp.float32)

    full = pl.BlockSpec((D, D), lambda i: (0, 0))
    return pl.pallas_call(
        body,
        grid=(N // _BLK,),
        in_specs=[pl.BlockSpec((NC, _BLK, DH), lambda i: (0, i, 0)),
                  pl.BlockSpec((_BLK, DW), lambda i: (i, 0)),
                  pl.BlockSpec((_BLK, D), lambda i: (i, 0)),
                  full, full,
                  pl.BlockSpec((2, D), lambda i: (0, 0))],
        out_specs=pl.BlockSpec((_BLK, DW), lambda i: (i, 0)),
        out_shape=jax.ShapeDtypeStruct((N, DW), jnp.float32),
    )(s1, cnt, u, W2l, W2r, Wlin)


def _tc_post(s2, cnt, t, Wlin, b2l, b2r, blin):
    def body(s_ref, c_ref, t_ref, wlin_ref, bl_ref, br_ref, blin_ref, o_ref):
        s = s_ref[0] + s_ref[1]
        cntv = jnp.maximum(c_ref[:, 0:1], 1.0)
        agg = s[:, 0:2] / cntv
        b2 = bl_ref[...] + br_ref[...]
        cconst = lax.dot_general(b2, wlin_ref[...], (((1,), (1,)), ((), ())),
                                 preferred_element_type=jnp.float32)
        o_ref[...] = agg + t_ref[:, 2:4] + cconst + blin_ref[...]

    return pl.pallas_call(
        body,
        grid=(N // _BLK,),
        in_specs=[pl.BlockSpec((NC, _BLK, DW), lambda i: (0, i, 0)),
                  pl.BlockSpec((_BLK, DW), lambda i: (i, 0)),
                  pl.BlockSpec((_BLK, DW), lambda i: (i, 0)),
                  pl.BlockSpec((2, D), lambda i: (0, 0)),
                  pl.BlockSpec((1, D), lambda i: (0, 0)),
                  pl.BlockSpec((1, D), lambda i: (0, 0)),
                  pl.BlockSpec((1, 2), lambda i: (0, 0))],
        out_specs=pl.BlockSpec((_BLK, 2), lambda i: (i, 0)),
        out_shape=jax.ShapeDtypeStruct((N, 2), jnp.float32),
    )(s2, cnt, t, Wlin, b2l.reshape(1, D), b2r.reshape(1, D),
      blin.reshape(1, 2))


def kernel(x, edge_index, W1l, b1l, W1r, b1r, W2l, b2l, W2r, b2r, Wlin, blin):
    pad = E2 - E
    srcp = jnp.concatenate([edge_index[0], jnp.zeros((pad,), jnp.int32)])
    dstp = jnp.concatenate([edge_index[1], jnp.full((pad,), N, jnp.int32)])
    src1 = srcp.reshape(NS, NCH1, C)
    dst1 = dstp.reshape(NS, NCH1, C)
    src2 = srcp.reshape(NW, NCH2, C)
    dst2 = dstp.reshape(NW, NCH2, C)
    zh = jnp.zeros((NP, DH), jnp.float32)
    zw = jnp.zeros((NP, DW), jnp.float32)

    y, u = _tc_pre(x, W1l, W1r, b1l, b1r)
    s1, cnt = _sc_agg_wide(src1, dst1, y, zh, zw)
    t = _tc_mid(s1, cnt, u, W2l, W2r, Wlin)
    s2 = _sc_agg_narrow(src2, dst2, t, zw)
    return _tc_post(s2, cnt, t, Wlin, b2l, b2r, blin)


# speedup vs baseline: 1.2625x; 1.0179x over previous
import functools

import jax
import jax.numpy as jnp
from jax import lax
from jax.experimental import pallas as pl
from jax.experimental.pallas import tpu as pltpu
from jax.experimental.pallas import tpu_sc as plsc

N = 10000
E = 320000
D = 128
DH = 64
DW = 16
NC, NS = 2, 16
NW = NC * NS
C = 256
E2 = 327680
NCH1 = E2 // (NS * C)
NCH2 = E2 // (NW * C)
NP = 10112
RPS = NP // NS

_MESH = plsc.VectorSubcoreMesh(core_axis_name="c", subcore_axis_name="s")


@functools.partial(
    pl.kernel,
    out_type=[
        jax.ShapeDtypeStruct((NC, NP, DH), jnp.float32),
        jax.ShapeDtypeStruct((NC, NP, DW), jnp.float32),
    ],
    mesh=_MESH,
    compiler_params=pltpu.CompilerParams(use_tc_tiling_on_sc=False),
    scratch_types=[
        pltpu.VMEM((NCH1, C), jnp.int32),
        pltpu.VMEM((NCH1, C), jnp.int32),
        pltpu.VMEM((C, DH), jnp.float32),
        pltpu.VMEM((C, DH), jnp.float32),
        pltpu.VMEM((C, DW), jnp.float32),
        pltpu.VMEM_SHARED((NP, DH), jnp.float32),
        pltpu.VMEM_SHARED((NP, DW), jnp.float32),
        pltpu.SemaphoreType.DMA,
        pltpu.SemaphoreType.DMA,
        pltpu.SemaphoreType.DMA,
    ],
)
def _sc_agg_wide(src_hbm, dst_hbm, y_hbm, zh_hbm, zw_hbm,
                 sum_out, cnt_out,
                 src_v, dst_v, rows_a, rows_b, ones_v,
                 acc_sh, cnt_sh,
                 sem_a, sem_b, sem_c):
    cid = lax.axis_index("c")
    sid = lax.axis_index("s")

    pltpu.sync_copy(zh_hbm.at[pl.ds(sid * RPS, RPS)],
                    acc_sh.at[pl.ds(sid * RPS, RPS)])
    pltpu.sync_copy(zw_hbm.at[pl.ds(sid * RPS, RPS)],
                    cnt_sh.at[pl.ds(sid * RPS, RPS)])

    pltpu.sync_copy(src_hbm.at[sid], src_v)
    pltpu.sync_copy(dst_hbm.at[sid], dst_v)

    @pl.loop(0, C)
    def _fill_ones(i):
        ones_v[i, :] = jnp.ones((DW,), jnp.float32)

    plsc.subcore_barrier()

    pltpu.async_copy(y_hbm.at[cid].at[src_v.at[0]], rows_a, sem_a)

    @pl.loop(0, NCH1 // 2)
    def _pair(j):
        i0 = 2 * j
        i1 = 2 * j + 1
        db = pltpu.async_copy(y_hbm.at[cid].at[src_v.at[i1]], rows_b, sem_b)
        pltpu.make_async_copy(y_hbm.at[cid].at[src_v.at[i0]], rows_a,
                              sem_a).wait()
        pltpu.sync_copy(rows_a, acc_sh.at[dst_v.at[i0]], add=True)

        @pl.when(cid == 0)
        def _count0():
            pltpu.async_copy(ones_v, cnt_sh.at[dst_v.at[i0]], sem_c,
                             add=True)

        @pl.when(j < NCH1 // 2 - 1)
        def _next():
            pltpu.async_copy(y_hbm.at[cid].at[src_v.at[i0 + 2]], rows_a,
                             sem_a)

        db.wait()
        pltpu.sync_copy(rows_b, acc_sh.at[dst_v.at[i1]], add=True)

        @pl.when(cid == 1)
        def _count1():
            pltpu.async_copy(ones_v, cnt_sh.at[dst_v.at[i1]], sem_c,
                             add=True)

    @pl.loop(0, NCH1 // 2)
    def _drain(i):
        pltpu.make_async_copy(ones_v, cnt_sh.at[dst_v.at[0]], sem_c).wait()

    plsc.subcore_barrier()

    pltpu.sync_copy(acc_sh.at[pl.ds(sid * RPS, RPS)],
                    sum_out.at[cid, pl.ds(sid * RPS, RPS)])

    pltpu.sync_copy(cnt_sh.at[pl.ds(sid * RPS, RPS)],
                    cnt_out.at[cid, pl.ds(sid * RPS, RPS)])


@functools.partial(
    pl.kernel,
    out_type=jax.ShapeDtypeStruct((NC, NP, DW), jnp.float32),
    mesh=_MESH,
    compiler_params=pltpu.CompilerParams(use_tc_tiling_on_sc=False),
    scratch_types=[
        pltpu.VMEM((NCH2, C), jnp.int32),
        pltpu.VMEM((NCH2, C), jnp.int32),
        pltpu.VMEM((C, DW), jnp.float32),
        pltpu.VMEM((C, DW), jnp.float32),
        pltpu.VMEM_SHARED((NP, DW), jnp.float32),
        pltpu.SemaphoreType.DMA,
        pltpu.SemaphoreType.DMA,
    ],
)
def _sc_agg_narrow(src_hbm, dst_hbm, t_hbm, zw_hbm, sum_out,
                   src_v, dst_v, rows_a, rows_b, acc_sh, sem_a, sem_b):
    cid = lax.axis_index("c")
    sid = lax.axis_index("s")
    wid = cid * NS + sid

    pltpu.sync_copy(zw_hbm.at[pl.ds(sid * RPS, RPS)],
                    acc_sh.at[pl.ds(sid * RPS, RPS)])
    pltpu.sync_copy(src_hbm.at[wid], src_v)
    pltpu.sync_copy(dst_hbm.at[wid], dst_v)

    plsc.subcore_barrier()

    pltpu.async_copy(t_hbm.at[src_v.at[0]], rows_a, sem_a)

    @pl.loop(0, NCH2 // 2)
    def _pair(j):
        i0 = 2 * j
        i1 = 2 * j + 1
        db = pltpu.async_copy(t_hbm.at[src_v.at[i1]], rows_b, sem_b)
        pltpu.make_async_copy(t_hbm.at[src_v.at[i0]], rows_a, sem_a).wait()
        pltpu.sync_copy(rows_a, acc_sh.at[dst_v.at[i0]], add=True)

        @pl.when(j < NCH2 // 2 - 1)
        def _next():
            pltpu.async_copy(t_hbm.at[src_v.at[i0 + 2]], rows_a, sem_a)

        db.wait()
        pltpu.sync_copy(rows_b, acc_sh.at[dst_v.at[i1]], add=True)

    plsc.subcore_barrier()

    pltpu.sync_copy(acc_sh.at[pl.ds(sid * RPS, RPS)],
                    sum_out.at[cid, pl.ds(sid * RPS, RPS)])


_BLK = 1000


def _tc_pre(x, W1l, W1r, b1l, b1r):
    def body(x_ref, wl_ref, wr_ref, bl_ref, br_ref, y_ref, u_ref):
        xb = x_ref[...]
        y = lax.dot_general(xb, wl_ref[...], (((1,), (1,)), ((), ())),
                            preferred_element_type=jnp.float32)
        y_ref[0, :, :] = y[:, :DH]
        y_ref[1, :, :] = y[:, DH:]
        u_ref[...] = (lax.dot_general(xb, wr_ref[...], (((1,), (1,)), ((), ())),
                                      preferred_element_type=jnp.float32)
                      + bl_ref[...] + br_ref[...])

    full = pl.BlockSpec((D, D), lambda i: (0, 0))
    bias = pl.BlockSpec((1, D), lambda i: (0, 0))
    return pl.pallas_call(
        body,
        grid=(N // _BLK,),
        in_specs=[pl.BlockSpec((_BLK, D), lambda i: (i, 0)), full, full,
                  bias, bias],
        out_specs=[pl.BlockSpec((NC, _BLK, DH), lambda i: (0, i, 0)),
                   pl.BlockSpec((_BLK, D), lambda i: (i, 0))],
        out_shape=[jax.ShapeDtypeStruct((NC, N, DH), jnp.float32),
                   jax.ShapeDtypeStruct((N, D), jnp.float32)],
    )(x, W1l, W1r, b1l.reshape(1, D), b1r.reshape(1, D))


def _tc_mid(s1, cnt, u, W2l, W2r, Wlin):
    def body(s_ref, c_ref, u_ref, w2l_ref, w2r_ref, wlin_ref, t_ref):
        s = jnp.concatenate([s_ref[0], s_ref[1]], axis=1)
        cntv = jnp.maximum(c_ref[0, :, 0:1] + c_ref[1, :, 0:1], 1.0)
        h = jnp.maximum(s / cntv + u_ref[...], 0.0)
        wl = wlin_ref[...]
        m = lax.dot_general(wl, w2l_ref[...], (((1,), (0,)), ((), ())),
                            preferred_element_type=jnp.float32)
        r = lax.dot_general(wl, w2r_ref[...], (((1,), (0,)), ((), ())),
                            preferred_element_type=jnp.float32)
        g = jnp.concatenate([m, r, jnp.zeros((DW - 4, D), jnp.float32)], axis=0)
        t_ref[...] = lax.dot_general(h, g, (((1,), (1,)), ((), ())),
                                     preferred_element_type=jnp.float32)

    full = pl.BlockSpec((D, D), lambda i: (0, 0))
    return pl.pallas_call(
        body,
        grid=(N // _BLK,),
        in_specs=[pl.BlockSpec((NC, _BLK, DH), lambda i: (0, i, 0)),
                  pl.BlockSpec((NC, _BLK, DW), lambda i: (0, i, 0)),
                  pl.BlockSpec((_BLK, D), lambda i: (i, 0)),
                  full, full,
                  pl.BlockSpec((2, D), lambda i: (0, 0))],
        out_specs=pl.BlockSpec((_BLK, DW), lambda i: (i, 0)),
        out_shape=jax.ShapeDtypeStruct((N, DW), jnp.float32),
    )(s1, cnt, u, W2l, W2r, Wlin)


def _tc_post(s2, cnt, t, Wlin, b2l, b2r, blin):
    def body(s_ref, c_ref, t_ref, wlin_ref, bl_ref, br_ref, blin_ref, o_ref):
        s = s_ref[0] + s_ref[1]
        cntv = jnp.maximum(c_ref[0, :, 0:1] + c_ref[1, :, 0:1], 1.0)
        agg = s[:, 0:2] / cntv
        b2 = bl_ref[...] + br_ref[...]
        cconst = lax.dot_general(b2, wlin_ref[...], (((1,), (1,)), ((), ())),
                                 preferred_element_type=jnp.float32)
        o_ref[...] = agg + t_ref[:, 2:4] + cconst + blin_ref[...]

    return pl.pallas_call(
        body,
        grid=(N // _BLK,),
        in_specs=[pl.BlockSpec((NC, _BLK, DW), lambda i: (0, i, 0)),
                  pl.BlockSpec((NC, _BLK, DW), lambda i: (0, i, 0)),
                  pl.BlockSpec((_BLK, DW), lambda i: (i, 0)),
                  pl.BlockSpec((2, D), lambda i: (0, 0)),
                  pl.BlockSpec((1, D), lambda i: (0, 0)),
                  pl.BlockSpec((1, D), lambda i: (0, 0)),
                  pl.BlockSpec((1, 2), lambda i: (0, 0))],
        out_specs=pl.BlockSpec((_BLK, 2), lambda i: (i, 0)),
        out_shape=jax.ShapeDtypeStruct((N, 2), jnp.float32),
    )(s2, cnt, t, Wlin, b2l.reshape(1, D), b2r.reshape(1, D),
      blin.reshape(1, 2))


def kernel(x, edge_index, W1l, b1l, W1r, b1r, W2l, b2l, W2r, b2r, Wlin, blin):
    pad = E2 - E
    srcp = jnp.concatenate([edge_index[0], jnp.zeros((pad,), jnp.int32)])
    dstp = jnp.concatenate([edge_index[1], jnp.full((pad,), N, jnp.int32)])
    src1 = srcp.reshape(NS, NCH1, C)
    dst1 = dstp.reshape(NS, NCH1, C)
    src2 = srcp.reshape(NW, NCH2, C)
    dst2 = dstp.reshape(NW, NCH2, C)
    zh = jnp.zeros((NP, DH), jnp.float32)
    zw = jnp.zeros((NP, DW), jnp.float32)

    y, u = _tc_pre(x, W1l, W1r, b1l, b1r)
    s1, cnt = _sc_agg_wide(src1, dst1, y, zh, zw)
    t = _tc_mid(s1, cnt, u, W2l, W2r, Wlin)
    s2 = _sc_agg_narrow(src2, dst2, t, zw)
    return _tc_post(s2, cnt, t, Wlin, b2l, b2r, blin)


# speedup vs baseline: 1.6766x; 1.3280x over previous
import functools

import jax
import jax.numpy as jnp
from jax import lax
from jax.experimental import pallas as pl
from jax.experimental.pallas import tpu as pltpu
from jax.experimental.pallas import tpu_sc as plsc

N = 10000
E = 320000
D = 128
DH = 64
DW = 16
NC, NS = 2, 16
NW = NC * NS
C = 256
E2 = 327680
NCH1 = E2 // (NS * C)
NCH2 = E2 // (NW * C)
NP = 10112
RPS = NP // NS

_MESH = plsc.VectorSubcoreMesh(core_axis_name="c", subcore_axis_name="s")


@functools.partial(
    pl.kernel,
    out_type=[
        jax.ShapeDtypeStruct((NC, NP, DH), jnp.bfloat16),
        jax.ShapeDtypeStruct((NC, NP, DW), jnp.float32),
    ],
    mesh=_MESH,
    compiler_params=pltpu.CompilerParams(use_tc_tiling_on_sc=False),
    scratch_types=[
        pltpu.VMEM((NCH1, C), jnp.int32),
        pltpu.VMEM((NCH1, C), jnp.int32),
        pltpu.VMEM((C, DH), jnp.bfloat16),
        pltpu.VMEM((C, DH), jnp.bfloat16),
        pltpu.VMEM((C, DW), jnp.float32),
        pltpu.VMEM_SHARED((NP, DH), jnp.bfloat16),
        pltpu.VMEM_SHARED((NP, DW), jnp.float32),
        pltpu.SemaphoreType.DMA,
        pltpu.SemaphoreType.DMA,
        pltpu.SemaphoreType.DMA,
    ],
)
def _sc_agg_wide(src_hbm, dst_hbm, y_hbm, zh_hbm, zw_hbm,
                 sum_out, cnt_out,
                 src_v, dst_v, rows_a, rows_b, ones_v,
                 acc_sh, cnt_sh,
                 sem_a, sem_b, sem_c):
    cid = lax.axis_index("c")
    sid = lax.axis_index("s")

    pltpu.sync_copy(zh_hbm.at[pl.ds(sid * RPS, RPS)],
                    acc_sh.at[pl.ds(sid * RPS, RPS)])
    pltpu.sync_copy(zw_hbm.at[pl.ds(sid * RPS, RPS)],
                    cnt_sh.at[pl.ds(sid * RPS, RPS)])

    pltpu.sync_copy(src_hbm.at[sid], src_v)
    pltpu.sync_copy(dst_hbm.at[sid], dst_v)

    @pl.loop(0, C)
    def _fill_ones(i):
        ones_v[i, :] = jnp.ones((DW,), jnp.float32)

    plsc.subcore_barrier()

    pltpu.async_copy(y_hbm.at[cid].at[src_v.at[0]], rows_a, sem_a)

    @pl.loop(0, NCH1 // 2)
    def _pair(j):
        i0 = 2 * j
        i1 = 2 * j + 1
        db = pltpu.async_copy(y_hbm.at[cid].at[src_v.at[i1]], rows_b, sem_b)
        pltpu.make_async_copy(y_hbm.at[cid].at[src_v.at[i0]], rows_a,
                              sem_a).wait()
        pltpu.sync_copy(rows_a, acc_sh.at[dst_v.at[i0]], add=True)

        @pl.when(cid == 0)
        def _count0():
            pltpu.async_copy(ones_v, cnt_sh.at[dst_v.at[i0]], sem_c,
                             add=True)

        @pl.when(j < NCH1 // 2 - 1)
        def _next():
            pltpu.async_copy(y_hbm.at[cid].at[src_v.at[i0 + 2]], rows_a,
                             sem_a)

        db.wait()
        pltpu.sync_copy(rows_b, acc_sh.at[dst_v.at[i1]], add=True)

        @pl.when(cid == 1)
        def _count1():
            pltpu.async_copy(ones_v, cnt_sh.at[dst_v.at[i1]], sem_c,
                             add=True)

    @pl.loop(0, NCH1 // 2)
    def _drain(i):
        pltpu.make_async_copy(ones_v, cnt_sh.at[dst_v.at[0]], sem_c).wait()

    plsc.subcore_barrier()

    pltpu.sync_copy(acc_sh.at[pl.ds(sid * RPS, RPS)],
                    sum_out.at[cid, pl.ds(sid * RPS, RPS)])

    pltpu.sync_copy(cnt_sh.at[pl.ds(sid * RPS, RPS)],
                    cnt_out.at[cid, pl.ds(sid * RPS, RPS)])


@functools.partial(
    pl.kernel,
    out_type=jax.ShapeDtypeStruct((NC, NP, DW), jnp.float32),
    mesh=_MESH,
    compiler_params=pltpu.CompilerParams(use_tc_tiling_on_sc=False),
    scratch_types=[
        pltpu.VMEM((NCH2, C), jnp.int32),
        pltpu.VMEM((NCH2, C), jnp.int32),
        pltpu.VMEM((C, DW), jnp.float32),
        pltpu.VMEM((C, DW), jnp.float32),
        pltpu.VMEM_SHARED((NP, DW), jnp.float32),
        pltpu.SemaphoreType.DMA,
        pltpu.SemaphoreType.DMA,
    ],
)
def _sc_agg_narrow(src_hbm, dst_hbm, t_hbm, zw_hbm, sum_out,
                   src_v, dst_v, rows_a, rows_b, acc_sh, sem_a, sem_b):
    cid = lax.axis_index("c")
    sid = lax.axis_index("s")
    wid = cid * NS + sid

    pltpu.sync_copy(zw_hbm.at[pl.ds(sid * RPS, RPS)],
                    acc_sh.at[pl.ds(sid * RPS, RPS)])
    pltpu.sync_copy(src_hbm.at[wid], src_v)
    pltpu.sync_copy(dst_hbm.at[wid], dst_v)

    plsc.subcore_barrier()

    pltpu.async_copy(t_hbm.at[src_v.at[0]], rows_a, sem_a)

    @pl.loop(0, NCH2 // 2)
    def _pair(j):
        i0 = 2 * j
        i1 = 2 * j + 1
        db = pltpu.async_copy(t_hbm.at[src_v.at[i1]], rows_b, sem_b)
        pltpu.make_async_copy(t_hbm.at[src_v.at[i0]], rows_a, sem_a).wait()
        pltpu.sync_copy(rows_a, acc_sh.at[dst_v.at[i0]], add=True)

        @pl.when(j < NCH2 // 2 - 1)
        def _next():
            pltpu.async_copy(t_hbm.at[src_v.at[i0 + 2]], rows_a, sem_a)

        db.wait()
        pltpu.sync_copy(rows_b, acc_sh.at[dst_v.at[i1]], add=True)

    plsc.subcore_barrier()

    pltpu.sync_copy(acc_sh.at[pl.ds(sid * RPS, RPS)],
                    sum_out.at[cid, pl.ds(sid * RPS, RPS)])


_BLK = 1000


def _tc_pre(x, W1l, W1r, b1l, b1r):
    def body(x_ref, wl_ref, wr_ref, bl_ref, br_ref, y_ref, u_ref):
        xb = x_ref[...]
        y = lax.dot_general(xb, wl_ref[...], (((1,), (1,)), ((), ())),
                            preferred_element_type=jnp.float32).astype(jnp.bfloat16)
        y_ref[0, :, :] = y[:, :DH]
        y_ref[1, :, :] = y[:, DH:]
        u_ref[...] = (lax.dot_general(xb, wr_ref[...], (((1,), (1,)), ((), ())),
                                      preferred_element_type=jnp.float32)
                      + bl_ref[...] + br_ref[...])

    full = pl.BlockSpec((D, D), lambda i: (0, 0))
    bias = pl.BlockSpec((1, D), lambda i: (0, 0))
    return pl.pallas_call(
        body,
        grid=(N // _BLK,),
        in_specs=[pl.BlockSpec((_BLK, D), lambda i: (i, 0)), full, full,
                  bias, bias],
        out_specs=[pl.BlockSpec((NC, _BLK, DH), lambda i: (0, i, 0)),
                   pl.BlockSpec((_BLK, D), lambda i: (i, 0))],
        out_shape=[jax.ShapeDtypeStruct((NC, N, DH), jnp.bfloat16),
                   jax.ShapeDtypeStruct((N, D), jnp.float32)],
    )(x, W1l, W1r, b1l.reshape(1, D), b1r.reshape(1, D))


def _tc_mid(s1, cnt, u, W2l, W2r, Wlin):
    def body(s_ref, c_ref, u_ref, w2l_ref, w2r_ref, wlin_ref, t_ref):
        s = jnp.concatenate([s_ref[0], s_ref[1]], axis=1).astype(jnp.float32)
        cntv = jnp.maximum(c_ref[0, :, 0:1] + c_ref[1, :, 0:1], 1.0)
        h = jnp.maximum(s / cntv + u_ref[...], 0.0)
        wl = wlin_ref[...]
        m = lax.dot_general(wl, w2l_ref[...], (((1,), (0,)), ((), ())),
                            preferred_element_type=jnp.float32)
        r = lax.dot_general(wl, w2r_ref[...], (((1,), (0,)), ((), ())),
                            preferred_element_type=jnp.float32)
        g = jnp.concatenate([m, r, jnp.zeros((DW - 4, D), jnp.float32)], axis=0)
        t_ref[...] = lax.dot_general(h, g, (((1,), (1,)), ((), ())),
                                     preferred_element_type=jnp.float32)

    full = pl.BlockSpec((D, D), lambda i: (0, 0))
    return pl.pallas_call(
        body,
        grid=(N // _BLK,),
        in_specs=[pl.BlockSpec((NC, _BLK, DH), lambda i: (0, i, 0)),
                  pl.BlockSpec((NC, _BLK, DW), lambda i: (0, i, 0)),
                  pl.BlockSpec((_BLK, D), lambda i: (i, 0)),
                  full, full,
                  pl.BlockSpec((2, D), lambda i: (0, 0))],
        out_specs=pl.BlockSpec((_BLK, DW), lambda i: (i, 0)),
        out_shape=jax.ShapeDtypeStruct((N, DW), jnp.float32),
    )(s1, cnt, u, W2l, W2r, Wlin)


def _tc_post(s2, cnt, t, Wlin, b2l, b2r, blin):
    def body(s_ref, c_ref, t_ref, wlin_ref, bl_ref, br_ref, blin_ref, o_ref):
        s = s_ref[0] + s_ref[1]
        cntv = jnp.maximum(c_ref[0, :, 0:1] + c_ref[1, :, 0:1], 1.0)
        agg = s[:, 0:2] / cntv
        b2 = bl_ref[...] + br_ref[...]
        cconst = lax.dot_general(b2, wlin_ref[...], (((1,), (1,)), ((), ())),
                                 preferred_element_type=jnp.float32)
        o_ref[...] = agg + t_ref[:, 2:4] + cconst + blin_ref[...]

    return pl.pallas_call(
        body,
        grid=(N // _BLK,),
        in_specs=[pl.BlockSpec((NC, _BLK, DW), lambda i: (0, i, 0)),
                  pl.BlockSpec((NC, _BLK, DW), lambda i: (0, i, 0)),
                  pl.BlockSpec((_BLK, DW), lambda i: (i, 0)),
                  pl.BlockSpec((2, D), lambda i: (0, 0)),
                  pl.BlockSpec((1, D), lambda i: (0, 0)),
                  pl.BlockSpec((1, D), lambda i: (0, 0)),
                  pl.BlockSpec((1, 2), lambda i: (0, 0))],
        out_specs=pl.BlockSpec((_BLK, 2), lambda i: (i, 0)),
        out_shape=jax.ShapeDtypeStruct((N, 2), jnp.float32),
    )(s2, cnt, t, Wlin, b2l.reshape(1, D), b2r.reshape(1, D),
      blin.reshape(1, 2))


def kernel(x, edge_index, W1l, b1l, W1r, b1r, W2l, b2l, W2r, b2r, Wlin, blin):
    pad = E2 - E
    srcp = jnp.concatenate([edge_index[0], jnp.zeros((pad,), jnp.int32)])
    dstp = jnp.concatenate([edge_index[1], jnp.full((pad,), N, jnp.int32)])
    src1 = srcp.reshape(NS, NCH1, C)
    dst1 = dstp.reshape(NS, NCH1, C)
    src2 = srcp.reshape(NW, NCH2, C)
    dst2 = dstp.reshape(NW, NCH2, C)
    zh = jnp.zeros((NP, DH), jnp.bfloat16)
    zw = jnp.zeros((NP, DW), jnp.float32)

    y, u = _tc_pre(x, W1l, W1r, b1l, b1r)
    s1, cnt = _sc_agg_wide(src1, dst1, y, zh, zw)
    t = _tc_mid(s1, cnt, u, W2l, W2r, Wlin)
    s2 = _sc_agg_narrow(src2, dst2, t, zw)
    return _tc_post(s2, cnt, t, Wlin, b2l, b2r, blin)


# speedup vs baseline: 1.8966x; 1.1312x over previous
import functools

import jax
import jax.numpy as jnp
from jax import lax
from jax.experimental import pallas as pl
from jax.experimental.pallas import tpu as pltpu
from jax.experimental.pallas import tpu_sc as plsc

N = 10000
E = 320000
D = 128
DH = 64
DW = 16
NC, NS = 2, 16
NW = NC * NS
C = 256
E2 = 327680
NCH1 = E2 // (NS * C)
NCH2 = E2 // (NW * C)
NP = 10112
RPS = NP // NS

_MESH = plsc.VectorSubcoreMesh(core_axis_name="c", subcore_axis_name="s")


@functools.partial(
    pl.kernel,
    out_type=[
        jax.ShapeDtypeStruct((NC, NP, DH), jnp.bfloat16),
        jax.ShapeDtypeStruct((NC, NP, DW), jnp.float32),
    ],
    mesh=_MESH,
    compiler_params=pltpu.CompilerParams(use_tc_tiling_on_sc=False),
    scratch_types=[
        pltpu.VMEM((NCH1, C), jnp.int32),
        pltpu.VMEM((NCH1, C), jnp.int32),
        pltpu.VMEM((C, DH), jnp.bfloat16),
        pltpu.VMEM((C, DH), jnp.bfloat16),
        pltpu.VMEM((C, DW), jnp.float32),
        pltpu.VMEM_SHARED((NP, DH), jnp.bfloat16),
        pltpu.VMEM_SHARED((NP, DW), jnp.float32),
        pltpu.SemaphoreType.DMA,
        pltpu.SemaphoreType.DMA,
        pltpu.SemaphoreType.DMA,
    ],
)
def _sc_agg_wide(src_hbm, dst_hbm, y_hbm, zh_hbm, zw_hbm,
                 sum_out, cnt_out,
                 src_v, dst_v, rows_a, rows_b, ones_v,
                 acc_sh, cnt_sh,
                 sem_a, sem_b, sem_c):
    cid = lax.axis_index("c")
    sid = lax.axis_index("s")

    pltpu.sync_copy(zh_hbm.at[pl.ds(sid * RPS, RPS)],
                    acc_sh.at[pl.ds(sid * RPS, RPS)])
    pltpu.sync_copy(zw_hbm.at[pl.ds(sid * RPS, RPS)],
                    cnt_sh.at[pl.ds(sid * RPS, RPS)])

    pltpu.sync_copy(src_hbm.at[sid], src_v)
    pltpu.sync_copy(dst_hbm.at[sid], dst_v)

    @pl.loop(0, C)
    def _fill_ones(i):
        ones_v[i, :] = jnp.ones((DW,), jnp.float32)

    plsc.subcore_barrier()

    pltpu.async_copy(y_hbm.at[cid].at[src_v.at[0]], rows_a, sem_a)

    @pl.loop(0, NCH1 // 2)
    def _pair(j):
        i0 = 2 * j
        i1 = 2 * j + 1
        db = pltpu.async_copy(y_hbm.at[cid].at[src_v.at[i1]], rows_b, sem_b)
        pltpu.make_async_copy(y_hbm.at[cid].at[src_v.at[i0]], rows_a,
                              sem_a).wait()
        pltpu.sync_copy(rows_a, acc_sh.at[dst_v.at[i0]], add=True)

        @pl.when(cid == 0)
        def _count0():
            pltpu.async_copy(ones_v, cnt_sh.at[dst_v.at[i0]], sem_c,
                             add=True)

        @pl.when(j < NCH1 // 2 - 1)
        def _next():
            pltpu.async_copy(y_hbm.at[cid].at[src_v.at[i0 + 2]], rows_a,
                             sem_a)

        db.wait()
        pltpu.sync_copy(rows_b, acc_sh.at[dst_v.at[i1]], add=True)

        @pl.when(cid == 1)
        def _count1():
            pltpu.async_copy(ones_v, cnt_sh.at[dst_v.at[i1]], sem_c,
                             add=True)

    @pl.loop(0, NCH1 // 2)
    def _drain(i):
        pltpu.make_async_copy(ones_v, cnt_sh.at[dst_v.at[0]], sem_c).wait()

    plsc.subcore_barrier()

    pltpu.sync_copy(acc_sh.at[pl.ds(sid * RPS, RPS)],
                    sum_out.at[cid, pl.ds(sid * RPS, RPS)])

    pltpu.sync_copy(cnt_sh.at[pl.ds(sid * RPS, RPS)],
                    cnt_out.at[cid, pl.ds(sid * RPS, RPS)])


@functools.partial(
    pl.kernel,
    out_type=jax.ShapeDtypeStruct((NC, NP, DW), jnp.bfloat16),
    mesh=_MESH,
    compiler_params=pltpu.CompilerParams(use_tc_tiling_on_sc=False),
    scratch_types=[
        pltpu.VMEM((NCH2, C), jnp.int32),
        pltpu.VMEM((NCH2, C), jnp.int32),
        pltpu.VMEM((C, DW), jnp.bfloat16),
        pltpu.VMEM((C, DW), jnp.bfloat16),
        pltpu.VMEM_SHARED((NP, DW), jnp.bfloat16),
        pltpu.SemaphoreType.DMA,
        pltpu.SemaphoreType.DMA,
    ],
)
def _sc_agg_narrow(src_hbm, dst_hbm, t_hbm, zw_hbm, sum_out,
                   src_v, dst_v, rows_a, rows_b, acc_sh, sem_a, sem_b):
    cid = lax.axis_index("c")
    sid = lax.axis_index("s")
    wid = cid * NS + sid

    pltpu.sync_copy(zw_hbm.at[pl.ds(sid * RPS, RPS)],
                    acc_sh.at[pl.ds(sid * RPS, RPS)])
    pltpu.sync_copy(src_hbm.at[wid], src_v)
    pltpu.sync_copy(dst_hbm.at[wid], dst_v)

    plsc.subcore_barrier()

    pltpu.async_copy(t_hbm.at[src_v.at[0]], rows_a, sem_a)

    @pl.loop(0, NCH2 // 2)
    def _pair(j):
        i0 = 2 * j
        i1 = 2 * j + 1
        db = pltpu.async_copy(t_hbm.at[src_v.at[i1]], rows_b, sem_b)
        pltpu.make_async_copy(t_hbm.at[src_v.at[i0]], rows_a, sem_a).wait()
        pltpu.sync_copy(rows_a, acc_sh.at[dst_v.at[i0]], add=True)

        @pl.when(j < NCH2 // 2 - 1)
        def _next():
            pltpu.async_copy(t_hbm.at[src_v.at[i0 + 2]], rows_a, sem_a)

        db.wait()
        pltpu.sync_copy(rows_b, acc_sh.at[dst_v.at[i1]], add=True)

    plsc.subcore_barrier()

    pltpu.sync_copy(acc_sh.at[pl.ds(sid * RPS, RPS)],
                    sum_out.at[cid, pl.ds(sid * RPS, RPS)])


_BLK = 1000


def _tc_pre(x, W1l, W1r, b1l, b1r):
    def body(x_ref, wl_ref, wr_ref, bl_ref, br_ref, y_ref, u_ref):
        xb = x_ref[...]
        y = lax.dot_general(xb, wl_ref[...], (((1,), (1,)), ((), ())),
                            preferred_element_type=jnp.float32).astype(jnp.bfloat16)
        y_ref[0, :, :] = y[:, :DH]
        y_ref[1, :, :] = y[:, DH:]
        u_ref[...] = (lax.dot_general(xb, wr_ref[...], (((1,), (1,)), ((), ())),
                                      preferred_element_type=jnp.float32)
                      + bl_ref[...] + br_ref[...])

    full = pl.BlockSpec((D, D), lambda i: (0, 0))
    bias = pl.BlockSpec((1, D), lambda i: (0, 0))
    return pl.pallas_call(
        body,
        grid=(N // _BLK,),
        in_specs=[pl.BlockSpec((_BLK, D), lambda i: (i, 0)), full, full,
                  bias, bias],
        out_specs=[pl.BlockSpec((NC, _BLK, DH), lambda i: (0, i, 0)),
                   pl.BlockSpec((_BLK, D), lambda i: (i, 0))],
        out_shape=[jax.ShapeDtypeStruct((NC, N, DH), jnp.bfloat16),
                   jax.ShapeDtypeStruct((N, D), jnp.float32)],
    )(x, W1l, W1r, b1l.reshape(1, D), b1r.reshape(1, D))


def _tc_mid(s1, cnt, u, W2l, W2r, Wlin):
    def body(s_ref, c_ref, u_ref, w2l_ref, w2r_ref, wlin_ref, t_ref, tb_ref):
        s = jnp.concatenate([s_ref[0], s_ref[1]], axis=1).astype(jnp.float32)
        cntv = jnp.maximum(c_ref[0, :, 0:1] + c_ref[1, :, 0:1], 1.0)
        h = jnp.maximum(s / cntv + u_ref[...], 0.0)
        wl = wlin_ref[...]
        m = lax.dot_general(wl, w2l_ref[...], (((1,), (0,)), ((), ())),
                            preferred_element_type=jnp.float32)
        r = lax.dot_general(wl, w2r_ref[...], (((1,), (0,)), ((), ())),
                            preferred_element_type=jnp.float32)
        g = jnp.concatenate([m, r, jnp.zeros((DW - 4, D), jnp.float32)], axis=0)
        t = lax.dot_general(h, g, (((1,), (1,)), ((), ())),
                            preferred_element_type=jnp.float32)
        t_ref[...] = t
        tb_ref[...] = t.astype(jnp.bfloat16)

    full = pl.BlockSpec((D, D), lambda i: (0, 0))
    return pl.pallas_call(
        body,
        grid=(N // _BLK,),
        in_specs=[pl.BlockSpec((NC, _BLK, DH), lambda i: (0, i, 0)),
                  pl.BlockSpec((NC, _BLK, DW), lambda i: (0, i, 0)),
                  pl.BlockSpec((_BLK, D), lambda i: (i, 0)),
                  full, full,
                  pl.BlockSpec((2, D), lambda i: (0, 0))],
        out_specs=[pl.BlockSpec((_BLK, DW), lambda i: (i, 0)),
                   pl.BlockSpec((_BLK, DW), lambda i: (i, 0))],
        out_shape=[jax.ShapeDtypeStruct((N, DW), jnp.float32),
                   jax.ShapeDtypeStruct((N, DW), jnp.bfloat16)],
    )(s1, cnt, u, W2l, W2r, Wlin)


def _tc_post(s2, cnt, t, Wlin, b2l, b2r, blin):
    def body(s_ref, c_ref, t_ref, wlin_ref, bl_ref, br_ref, blin_ref, o_ref):
        s = s_ref[0].astype(jnp.float32) + s_ref[1].astype(jnp.float32)
        cntv = jnp.maximum(c_ref[0, :, 0:1] + c_ref[1, :, 0:1], 1.0)
        agg = s[:, 0:2] / cntv
        b2 = bl_ref[...] + br_ref[...]
        cconst = lax.dot_general(b2, wlin_ref[...], (((1,), (1,)), ((), ())),
                                 preferred_element_type=jnp.float32)
        o_ref[...] = agg + t_ref[:, 2:4] + cconst + blin_ref[...]

    return pl.pallas_call(
        body,
        grid=(N // _BLK,),
        in_specs=[pl.BlockSpec((NC, _BLK, DW), lambda i: (0, i, 0)),
                  pl.BlockSpec((NC, _BLK, DW), lambda i: (0, i, 0)),
                  pl.BlockSpec((_BLK, DW), lambda i: (i, 0)),
                  pl.BlockSpec((2, D), lambda i: (0, 0)),
                  pl.BlockSpec((1, D), lambda i: (0, 0)),
                  pl.BlockSpec((1, D), lambda i: (0, 0)),
                  pl.BlockSpec((1, 2), lambda i: (0, 0))],
        out_specs=pl.BlockSpec((_BLK, 2), lambda i: (i, 0)),
        out_shape=jax.ShapeDtypeStruct((N, 2), jnp.float32),
    )(s2, cnt, t, Wlin, b2l.reshape(1, D), b2r.reshape(1, D),
      blin.reshape(1, 2))


def kernel(x, edge_index, W1l, b1l, W1r, b1r, W2l, b2l, W2r, b2r, Wlin, blin):
    pad = E2 - E
    srcp = jnp.concatenate([edge_index[0], jnp.zeros((pad,), jnp.int32)])
    dstp = jnp.concatenate([edge_index[1], jnp.full((pad,), N, jnp.int32)])
    src1 = srcp.reshape(NS, NCH1, C)
    dst1 = dstp.reshape(NS, NCH1, C)
    src2 = srcp.reshape(NW, NCH2, C)
    dst2 = dstp.reshape(NW, NCH2, C)
    zh = jnp.zeros((NP, DH), jnp.bfloat16)
    zw = jnp.zeros((NP, DW), jnp.float32)
    zwb = jnp.zeros((NP, DW), jnp.bfloat16)

    y, u = _tc_pre(x, W1l, W1r, b1l, b1r)
    s1, cnt = _sc_agg_wide(src1, dst1, y, zh, zw)
    t, tb = _tc_mid(s1, cnt, u, W2l, W2r, Wlin)
    s2 = _sc_agg_narrow(src2, dst2, tb, zwb)
    return _tc_post(s2, cnt, t, Wlin, b2l, b2r, blin)
